# baseline (speedup 1.0000x reference)
import functools

import jax
import jax.numpy as jnp
from jax import lax
from jax.experimental import pallas as pl
from jax.experimental.pallas import tpu as pltpu
from jax.experimental.pallas import tpu_sc as plsc

N_NODES = 50000
N_EDGES = 800000
HID = 64
HALF = 32
D1 = 16
NG = 512
NC = 2
NS = 16
BLK = 2000
NB = N_NODES // BLK
EPS = 1e-5


def _sc_edge_loop(tab, acc, ep_h, idx0, idx1, rows0, rows1, sem0, sem1,
                  cbase, nch):
    CH = idx0.shape[1]

    def load(i, idx):
        pltpu.sync_copy(ep_h.at[0, pl.ds(i * CH, CH)], idx.at[0])
        pltpu.sync_copy(ep_h.at[1, pl.ds(i * CH, CH)], idx.at[1])

    load(cbase, idx0)
    pltpu.async_copy(tab.at[idx0.at[0]], rows0, sem0)
    load(cbase + 1, idx1)
    pltpu.async_copy(tab.at[idx1.at[0]], rows1, sem1)

    def pair(g, carry):
        i0 = cbase + 2 * g
        pltpu.make_async_copy(tab.at[idx0.at[0]], rows0, sem0).wait()
        pltpu.sync_copy(rows0, acc.at[idx0.at[1]], add=True)

        @pl.when(2 * g + 2 < nch)
        def _():
            load(i0 + 2, idx0)
            pltpu.async_copy(tab.at[idx0.at[0]], rows0, sem0)

        @pl.when(2 * g + 1 < nch)
        def _():
            pltpu.make_async_copy(tab.at[idx1.at[0]], rows1, sem1).wait()
            pltpu.sync_copy(rows1, acc.at[idx1.at[1]], add=True)

            @pl.when(2 * g + 3 < nch)
            def _():
                load(i0 + 3, idx1)
                pltpu.async_copy(tab.at[idx1.at[0]], rows1, sem1)

        return carry

    lax.fori_loop(0, (nch + 1) // 2, pair, 0)


def _sc_agg(h_lo, h_hi, ep):
    N, D = h_lo.shape
    E = ep.shape[1]
    CH = 400
    NCH = E // CH // NS
    RPT = N // NS

    mesh = plsc.VectorSubcoreMesh(
        core_axis_name="c", subcore_axis_name="s", num_cores=NC, num_subcores=NS
    )

    @functools.partial(
        pl.kernel,
        out_type=(
            jax.ShapeDtypeStruct((N, D), jnp.float32),
            jax.ShapeDtypeStruct((N, D), jnp.float32),
        ),
        mesh=mesh,
        compiler_params=pltpu.CompilerParams(use_tc_tiling_on_sc=False),
        scratch_types=[
            pltpu.VMEM_SHARED((N, D), jnp.float32),
            pltpu.VMEM((2, CH), jnp.int32),
            pltpu.VMEM((2, CH), jnp.int32),
            pltpu.VMEM((CH, D), jnp.float32),
            pltpu.VMEM((CH, D), jnp.float32),
            pltpu.SemaphoreType.DMA,
            pltpu.SemaphoreType.DMA,
        ],
    )
    def k(hlo, hhi, ep_h, olo, ohi,
          acc, idx0, idx1, rows0, rows1, sem0, sem1):
        c = lax.axis_index("c")
        s = lax.axis_index("s")

        def run(tab, out):
            pltpu.sync_copy(tab.at[pl.ds(s * RPT, RPT)],
                            acc.at[pl.ds(s * RPT, RPT)])
            plsc.subcore_barrier()
            _sc_edge_loop(tab, acc, ep_h, idx0, idx1, rows0, rows1,
                          sem0, sem1, s * NCH, NCH)
            plsc.subcore_barrier()
            pltpu.sync_copy(acc.at[pl.ds(s * RPT, RPT)],
                            out.at[pl.ds(s * RPT, RPT)])

        @pl.when(c == 0)
        def _():
            run(hlo, olo)

        @pl.when(c == 1)
        def _():
            run(hhi, ohi)

    return k(h_lo, h_hi, ep)


def _sc_agg1(x16, z16, ep):
    N, D = x16.shape
    E = ep.shape[1]
    CH = 1000
    NCH = E // CH // (NC * NS)
    RPT = N // NS

    mesh = plsc.VectorSubcoreMesh(
        core_axis_name="c", subcore_axis_name="s", num_cores=NC, num_subcores=NS
    )

    @functools.partial(
        pl.kernel,
        out_type=(
            jax.ShapeDtypeStruct((N, D), jnp.float32),
            jax.ShapeDtypeStruct((N, D), jnp.float32),
        ),
        mesh=mesh,
        compiler_params=pltpu.CompilerParams(use_tc_tiling_on_sc=False),
        scratch_types=[
            pltpu.VMEM_SHARED((N, D), jnp.float32),
            pltpu.VMEM((2, CH), jnp.int32),
            pltpu.VMEM((2, CH), jnp.int32),
            pltpu.VMEM((CH, D), jnp.float32),
            pltpu.VMEM((CH, D), jnp.float32),
            pltpu.SemaphoreType.DMA,
            pltpu.SemaphoreType.DMA,
        ],
    )
    def k(x_h, z_h, ep_h, o0, o1,
          acc, idx0, idx1, rows0, rows1, sem0, sem1):
        c = lax.axis_index("c")
        s = lax.axis_index("s")

        def run(init_tab, out):
            pltpu.sync_copy(init_tab.at[pl.ds(s * RPT, RPT)],
                            acc.at[pl.ds(s * RPT, RPT)])
            plsc.subcore_barrier()
            _sc_edge_loop(x_h, acc, ep_h, idx0, idx1, rows0, rows1,
                          sem0, sem1, (c * NS + s) * NCH, NCH)
            plsc.subcore_barrier()
            pltpu.sync_copy(acc.at[pl.ds(s * RPT, RPT)],
                            out.at[pl.ds(s * RPT, RPT)])

        @pl.when(c == 0)
        def _():
            run(x_h, o0)

        @pl.when(c == 1)
        def _():
            run(z_h, o1)

    return k(x16, z16, ep)


def _bn_coeffs(st, gamma, beta):
    mean = st[0:1, :] / N_NODES
    var = st[1:2, :] / N_NODES - mean * mean
    rstd = gamma / jnp.sqrt(var + EPS)
    return rstd, beta - mean * rstd


def _tc_layer(s_lo, s_hi, W1, b1, W2, b2, gamma, beta, first):
    N = s_lo.shape[0]

    def body(lo_ref, hi_ref, w1_ref, b1_ref, w2_ref, b2_ref, g_ref, be_ref,
             olo_ref, ohi_ref, u_scr, st_scr):
        p = pl.program_id(0)
        j = pl.program_id(1)

        @pl.when(p == 0)
        def _():
            if first:
                s = (lo_ref[...] + hi_ref[...])[:, 0:2]
            else:
                s = jnp.concatenate([lo_ref[...], hi_ref[...]], axis=1)
            a = jnp.dot(s, w1_ref[...], preferred_element_type=jnp.float32)
            a = jnp.maximum(a + b1_ref[...], 0.0)
            b = jnp.dot(a, w2_ref[...], preferred_element_type=jnp.float32)
            b = jnp.maximum(b + b2_ref[...], 0.0)
            u_scr[pl.ds(j * BLK, BLK), :] = b

            @pl.when(j == 0)
            def _():
                st_scr[...] = jnp.zeros_like(st_scr)

            s0 = jnp.sum(b, axis=0, keepdims=True)
            s1 = jnp.sum(b * b, axis=0, keepdims=True)
            pad = jnp.zeros((6, HID), jnp.float32)
            st_scr[...] = st_scr[...] + jnp.concatenate([s0, s1, pad], axis=0)

        @pl.when(p == 1)
        def _():
            scale, shift = _bn_coeffs(st_scr[...], g_ref[...], be_ref[...])
            hn = u_scr[pl.ds(j * BLK, BLK), :] * scale + shift
            olo_ref[...] = hn[:, :HALF]
            ohi_ref[...] = hn[:, HALF:]

    din = s_lo.shape[1]
    return pl.pallas_call(
        body,
        grid=(2, NB),
        in_specs=[
            pl.BlockSpec((BLK, din), lambda p, j: (j * (1 - p), 0)),
            pl.BlockSpec((BLK, din), lambda p, j: (j * (1 - p), 0)),
            pl.BlockSpec(W1.shape, lambda p, j: (0, 0)),
            pl.BlockSpec((1, HID), lambda p, j: (0, 0)),
            pl.BlockSpec((HID, HID), lambda p, j: (0, 0)),
            pl.BlockSpec((1, HID), lambda p, j: (0, 0)),
            pl.BlockSpec((1, HID), lambda p, j: (0, 0)),
            pl.BlockSpec((1, HID), lambda p, j: (0, 0)),
        ],
        out_specs=[pl.BlockSpec((BLK, HALF), lambda p, j: (j * p, 0))] * 2,
        out_shape=[jax.ShapeDtypeStruct((N, HALF), jnp.float32)] * 2,
        scratch_shapes=[
            pltpu.VMEM((N, HID), jnp.float32),
            pltpu.VMEM((8, HID), jnp.float32),
        ],
    )(s_lo, s_hi, W1, b1, W2, b2, gamma, beta)


def _tc_layer_last(s_lo, s_hi, W1, b1, W2, b2, gamma, beta, batch3,
                   lin1_W, lin1_b, lin2_r):
    N = s_lo.shape[0]

    def body(lo_ref, hi_ref, w1_ref, b1_ref, w2_ref, b2_ref, g_ref, be_ref,
             b3_ref, lw_ref, lb_ref, l2_ref, out_ref, u_scr, st_scr, acc_ref):
        p = pl.program_id(0)
        j = pl.program_id(1)

        @pl.when(p == 0)
        def _():
            s = jnp.concatenate([lo_ref[...], hi_ref[...]], axis=1)
            a = jnp.dot(s, w1_ref[...], preferred_element_type=jnp.float32)
            a = jnp.maximum(a + b1_ref[...], 0.0)
            b = jnp.dot(a, w2_ref[...], preferred_element_type=jnp.float32)
            b = jnp.maximum(b + b2_ref[...], 0.0)
            u_scr[pl.ds(j * BLK, BLK), :] = b

            @pl.when(j == 0)
            def _():
                st_scr[...] = jnp.zeros_like(st_scr)

            s0 = jnp.sum(b, axis=0, keepdims=True)
            s1 = jnp.sum(b * b, axis=0, keepdims=True)
            pad = jnp.zeros((6, HID), jnp.float32)
            st_scr[...] = st_scr[...] + jnp.concatenate([s0, s1, pad], axis=0)

        @pl.when(p == 1)
        def _():
            scale, shift = _bn_coeffs(st_scr[...], g_ref[...], be_ref[...])
            hn = u_scr[pl.ds(j * BLK, BLK), :] * scale + shift
            bb = b3_ref[0, 0, :]
            seg = jax.lax.broadcasted_iota(jnp.int32, (BLK, NG), 1)
            onehot = (bb[:, None] == seg).astype(jnp.float32)
            part = jax.lax.dot_general(
                onehot, hn, (((0,), (0,)), ((), ())),
                preferred_element_type=jnp.float32,
                precision=jax.lax.Precision.HIGHEST)

            @pl.when(j == 0)
            def _():
                acc_ref[...] = jnp.zeros_like(acc_ref)

            acc_ref[...] = acc_ref[...] + part

            @pl.when(j == NB - 1)
            def _():
                pooled = acc_ref[...]
                r = jnp.dot(pooled, lw_ref[...],
                            preferred_element_type=jnp.float32)
                r = jnp.maximum(r + lb_ref[...], 0.0)
                o = jnp.sum(r * l2_ref[...], axis=1)
                out_ref[...] = o.reshape(1, NG)

    return pl.pallas_call(
        body,
        grid=(2, NB),
        in_specs=[
            pl.BlockSpec((BLK, HALF), lambda p, j: (j * (1 - p), 0)),
            pl.BlockSpec((BLK, HALF), lambda p, j: (j * (1 - p), 0)),
            pl.BlockSpec((HID, HID), lambda p, j: (0, 0)),
            pl.BlockSpec((1, HID), lambda p, j: (0, 0)),
            pl.BlockSpec((HID, HID), lambda p, j: (0, 0)),
            pl.BlockSpec((1, HID), lambda p, j: (0, 0)),
            pl.BlockSpec((1, HID), lambda p, j: (0, 0)),
            pl.BlockSpec((1, HID), lambda p, j: (0, 0)),
            pl.BlockSpec((1, 1, BLK), lambda p, j: (j * p, 0, 0)),
            pl.BlockSpec((HID, HID), lambda p, j: (0, 0)),
            pl.BlockSpec((1, HID), lambda p, j: (0, 0)),
            pl.BlockSpec((1, HID), lambda p, j: (0, 0)),
        ],
        out_specs=pl.BlockSpec((1, NG), lambda p, j: (0, 0)),
        out_shape=jax.ShapeDtypeStruct((1, NG), jnp.float32),
        scratch_shapes=[
            pltpu.VMEM((N, HID), jnp.float32),
            pltpu.VMEM((8, HID), jnp.float32),
            pltpu.VMEM((NG, HID), jnp.float32),
        ],
    )(s_lo, s_hi, W1, b1, W2, b2, gamma, beta, batch3, lin1_W, lin1_b, lin2_r)


def kernel(x, edge_index, batch, params):
    p1 = params["conv1"]

    x16 = jnp.pad(x, ((0, 0), (0, D1 - x.shape[1])))
    z16 = jnp.zeros_like(x16)
    batch3 = batch.reshape(NB, 1, BLK)

    layers = [p1] + list(params["convs"])
    out = None
    for li, p in enumerate(layers):
        args = (p["W1"], p["b1"].reshape(1, HID), p["W2"],
                p["b2"].reshape(1, HID), p["gamma"].reshape(1, HID),
                p["beta"].reshape(1, HID))
        if li == 0:
            a0, a1 = _sc_agg1(x16, z16, edge_index)
            h_lo, h_hi = _tc_layer(a0, a1, *args, first=True)
        elif li < len(layers) - 1:
            s_lo, s_hi = _sc_agg(h_lo, h_hi, edge_index)
            h_lo, h_hi = _tc_layer(s_lo, s_hi, *args, first=False)
        else:
            s_lo, s_hi = _sc_agg(h_lo, h_hi, edge_index)
            out = _tc_layer_last(s_lo, s_hi, *args, batch3,
                                 params["lin1_W"],
                                 params["lin1_b"].reshape(1, HID),
                                 params["lin2_W"].reshape(1, HID))

    return out.reshape(NG) + params["lin2_b"][0]


# speedup vs baseline: 1.0612x; 1.0612x over previous
import functools

import jax
import jax.numpy as jnp
from jax import lax
from jax.experimental import pallas as pl
from jax.experimental.pallas import tpu as pltpu
from jax.experimental.pallas import tpu_sc as plsc

N_NODES = 50000
N_EDGES = 800000
HID = 64
HALF = 32
D1 = 16
NG = 512
NC = 2
NS = 16
BLK = 2000
NB = N_NODES // BLK
EPS = 1e-5


def _sc_edge_loop(tab, acc, ep_h, idx0, idx1, rows0, rows1, sem0, sem1,
                  cbase, nch):
    pltpu.sync_copy(ep_h.at[cbase], idx0)
    pltpu.async_copy(tab.at[idx0.at[0]], rows0, sem0)
    pltpu.sync_copy(ep_h.at[cbase + 1], idx1)
    pltpu.async_copy(tab.at[idx1.at[0]], rows1, sem1)

    def pair(g, carry):
        i0 = cbase + 2 * g
        pltpu.make_async_copy(tab.at[idx0.at[0]], rows0, sem0).wait()
        pltpu.sync_copy(rows0, acc.at[idx0.at[1]], add=True)

        @pl.when(2 * g + 2 < nch)
        def _():
            pltpu.sync_copy(ep_h.at[i0 + 2], idx0)
            pltpu.async_copy(tab.at[idx0.at[0]], rows0, sem0)

        @pl.when(2 * g + 1 < nch)
        def _():
            pltpu.make_async_copy(tab.at[idx1.at[0]], rows1, sem1).wait()
            pltpu.sync_copy(rows1, acc.at[idx1.at[1]], add=True)

            @pl.when(2 * g + 3 < nch)
            def _():
                pltpu.sync_copy(ep_h.at[i0 + 3], idx1)
                pltpu.async_copy(tab.at[idx1.at[0]], rows1, sem1)

        return carry

    lax.fori_loop(0, (nch + 1) // 2, pair, 0)


def _sc_agg(h_lo, h_hi, ep):
    N, D = h_lo.shape
    NCHT, _, CH = ep.shape
    NCH = NCHT // NS
    RPT = N // NS

    mesh = plsc.VectorSubcoreMesh(
        core_axis_name="c", subcore_axis_name="s", num_cores=NC, num_subcores=NS
    )

    @functools.partial(
        pl.kernel,
        out_type=(
            jax.ShapeDtypeStruct((N, D), jnp.float32),
            jax.ShapeDtypeStruct((N, D), jnp.float32),
        ),
        mesh=mesh,
        compiler_params=pltpu.CompilerParams(use_tc_tiling_on_sc=False),
        scratch_types=[
            pltpu.VMEM_SHARED((N, D), jnp.float32),
            pltpu.VMEM((2, CH), jnp.int32),
            pltpu.VMEM((2, CH), jnp.int32),
            pltpu.VMEM((CH, D), jnp.float32),
            pltpu.VMEM((CH, D), jnp.float32),
            pltpu.SemaphoreType.DMA,
            pltpu.SemaphoreType.DMA,
        ],
    )
    def k(hlo, hhi, ep_h, olo, ohi,
          acc, idx0, idx1, rows0, rows1, sem0, sem1):
        c = lax.axis_index("c")
        s = lax.axis_index("s")

        def run(tab, out):
            pltpu.sync_copy(tab.at[pl.ds(s * RPT, RPT)],
                            acc.at[pl.ds(s * RPT, RPT)])
            plsc.subcore_barrier()
            _sc_edge_loop(tab, acc, ep_h, idx0, idx1, rows0, rows1,
                          sem0, sem1, s * NCH, NCH)
            plsc.subcore_barrier()
            pltpu.sync_copy(acc.at[pl.ds(s * RPT, RPT)],
                            out.at[pl.ds(s * RPT, RPT)])

        @pl.when(c == 0)
        def _():
            run(hlo, olo)

        @pl.when(c == 1)
        def _():
            run(hhi, ohi)

    return k(h_lo, h_hi, ep)


def _sc_agg1(x16, z16, ep):
    N, D = x16.shape
    NCHT, _, CH = ep.shape
    NCH = NCHT // (NC * NS)
    RPT = N // NS

    mesh = plsc.VectorSubcoreMesh(
        core_axis_name="c", subcore_axis_name="s", num_cores=NC, num_subcores=NS
    )

    @functools.partial(
        pl.kernel,
        out_type=(
            jax.ShapeDtypeStruct((N, D), jnp.float32),
            jax.ShapeDtypeStruct((N, D), jnp.float32),
        ),
        mesh=mesh,
        compiler_params=pltpu.CompilerParams(use_tc_tiling_on_sc=False),
        scratch_types=[
            pltpu.VMEM_SHARED((N, D), jnp.float32),
            pltpu.VMEM((2, CH), jnp.int32),
            pltpu.VMEM((2, CH), jnp.int32),
            pltpu.VMEM((CH, D), jnp.float32),
            pltpu.VMEM((CH, D), jnp.float32),
            pltpu.SemaphoreType.DMA,
            pltpu.SemaphoreType.DMA,
        ],
    )
    def k(x_h, z_h, ep_h, o0, o1,
          acc, idx0, idx1, rows0, rows1, sem0, sem1):
        c = lax.axis_index("c")
        s = lax.axis_index("s")

        def run(init_tab, out):
            pltpu.sync_copy(init_tab.at[pl.ds(s * RPT, RPT)],
                            acc.at[pl.ds(s * RPT, RPT)])
            plsc.subcore_barrier()
            _sc_edge_loop(x_h, acc, ep_h, idx0, idx1, rows0, rows1,
                          sem0, sem1, (c * NS + s) * NCH, NCH)
            plsc.subcore_barrier()
            pltpu.sync_copy(acc.at[pl.ds(s * RPT, RPT)],
                            out.at[pl.ds(s * RPT, RPT)])

        @pl.when(c == 0)
        def _():
            run(x_h, o0)

        @pl.when(c == 1)
        def _():
            run(z_h, o1)

    return k(x16, z16, ep)


def _bn_coeffs(st, gamma, beta):
    mean = st[0:1, :] / N_NODES
    var = st[1:2, :] / N_NODES - mean * mean
    rstd = gamma / jnp.sqrt(var + EPS)
    return rstd, beta - mean * rstd


def _tc_layer(s_lo, s_hi, W1, b1, W2, b2, gamma, beta, first):
    N = s_lo.shape[0]

    def body(lo_ref, hi_ref, w1_ref, b1_ref, w2_ref, b2_ref, g_ref, be_ref,
             olo_ref, ohi_ref, u_scr, st_scr):
        p = pl.program_id(0)
        j = pl.program_id(1)

        @pl.when(p == 0)
        def _():
            if first:
                s = (lo_ref[...] + hi_ref[...])[:, 0:2]
            else:
                s = jnp.concatenate([lo_ref[...], hi_ref[...]], axis=1)
            a = jnp.dot(s, w1_ref[...], preferred_element_type=jnp.float32)
            a = jnp.maximum(a + b1_ref[...], 0.0)
            b = jnp.dot(a, w2_ref[...], preferred_element_type=jnp.float32)
            b = jnp.maximum(b + b2_ref[...], 0.0)
            u_scr[pl.ds(j * BLK, BLK), :] = b

            @pl.when(j == 0)
            def _():
                st_scr[...] = jnp.zeros_like(st_scr)

            s0 = jnp.sum(b, axis=0, keepdims=True)
            s1 = jnp.sum(b * b, axis=0, keepdims=True)
            pad = jnp.zeros((6, HID), jnp.float32)
            st_scr[...] = st_scr[...] + jnp.concatenate([s0, s1, pad], axis=0)

        @pl.when(p == 1)
        def _():
            scale, shift = _bn_coeffs(st_scr[...], g_ref[...], be_ref[...])
            hn = u_scr[pl.ds(j * BLK, BLK), :] * scale + shift
            olo_ref[...] = hn[:, :HALF]
            ohi_ref[...] = hn[:, HALF:]

    din = s_lo.shape[1]
    return pl.pallas_call(
        body,
        grid=(2, NB),
        in_specs=[
            pl.BlockSpec((BLK, din), lambda p, j: (j * (1 - p), 0)),
            pl.BlockSpec((BLK, din), lambda p, j: (j * (1 - p), 0)),
            pl.BlockSpec(W1.shape, lambda p, j: (0, 0)),
            pl.BlockSpec((1, HID), lambda p, j: (0, 0)),
            pl.BlockSpec((HID, HID), lambda p, j: (0, 0)),
            pl.BlockSpec((1, HID), lambda p, j: (0, 0)),
            pl.BlockSpec((1, HID), lambda p, j: (0, 0)),
            pl.BlockSpec((1, HID), lambda p, j: (0, 0)),
        ],
        out_specs=[pl.BlockSpec((BLK, HALF), lambda p, j: (j * p, 0))] * 2,
        out_shape=[jax.ShapeDtypeStruct((N, HALF), jnp.float32)] * 2,
        scratch_shapes=[
            pltpu.VMEM((N, HID), jnp.float32),
            pltpu.VMEM((8, HID), jnp.float32),
        ],
    )(s_lo, s_hi, W1, b1, W2, b2, gamma, beta)


def _tc_layer_last(s_lo, s_hi, W1, b1, W2, b2, gamma, beta, batch3,
                   lin1_W, lin1_b, lin2_r):
    N = s_lo.shape[0]

    def body(lo_ref, hi_ref, w1_ref, b1_ref, w2_ref, b2_ref, g_ref, be_ref,
             b3_ref, lw_ref, lb_ref, l2_ref, out_ref, u_scr, st_scr, acc_ref):
        p = pl.program_id(0)
        j = pl.program_id(1)

        @pl.when(p == 0)
        def _():
            s = jnp.concatenate([lo_ref[...], hi_ref[...]], axis=1)
            a = jnp.dot(s, w1_ref[...], preferred_element_type=jnp.float32)
            a = jnp.maximum(a + b1_ref[...], 0.0)
            b = jnp.dot(a, w2_ref[...], preferred_element_type=jnp.float32)
            b = jnp.maximum(b + b2_ref[...], 0.0)
            u_scr[pl.ds(j * BLK, BLK), :] = b

            @pl.when(j == 0)
            def _():
                st_scr[...] = jnp.zeros_like(st_scr)

            s0 = jnp.sum(b, axis=0, keepdims=True)
            s1 = jnp.sum(b * b, axis=0, keepdims=True)
            pad = jnp.zeros((6, HID), jnp.float32)
            st_scr[...] = st_scr[...] + jnp.concatenate([s0, s1, pad], axis=0)

        @pl.when(p == 1)
        def _():
            scale, shift = _bn_coeffs(st_scr[...], g_ref[...], be_ref[...])
            hn = u_scr[pl.ds(j * BLK, BLK), :] * scale + shift
            bb = b3_ref[0, 0, :]
            seg = jax.lax.broadcasted_iota(jnp.int32, (BLK, NG), 1)
            onehot = (bb[:, None] == seg).astype(jnp.float32)
            part = jax.lax.dot_general(
                onehot, hn, (((0,), (0,)), ((), ())),
                preferred_element_type=jnp.float32,
                precision=jax.lax.Precision.HIGHEST)

            @pl.when(j == 0)
            def _():
                acc_ref[...] = jnp.zeros_like(acc_ref)

            acc_ref[...] = acc_ref[...] + part

            @pl.when(j == NB - 1)
            def _():
                pooled = acc_ref[...]
                r = jnp.dot(pooled, lw_ref[...],
                            preferred_element_type=jnp.float32)
                r = jnp.maximum(r + lb_ref[...], 0.0)
                o = jnp.sum(r * l2_ref[...], axis=1)
                out_ref[...] = o.reshape(1, NG)

    return pl.pallas_call(
        body,
        grid=(2, NB),
        in_specs=[
            pl.BlockSpec((BLK, HALF), lambda p, j: (j * (1 - p), 0)),
            pl.BlockSpec((BLK, HALF), lambda p, j: (j * (1 - p), 0)),
            pl.BlockSpec((HID, HID), lambda p, j: (0, 0)),
            pl.BlockSpec((1, HID), lambda p, j: (0, 0)),
            pl.BlockSpec((HID, HID), lambda p, j: (0, 0)),
            pl.BlockSpec((1, HID), lambda p, j: (0, 0)),
            pl.BlockSpec((1, HID), lambda p, j: (0, 0)),
            pl.BlockSpec((1, HID), lambda p, j: (0, 0)),
            pl.BlockSpec((1, 1, BLK), lambda p, j: (j * p, 0, 0)),
            pl.BlockSpec((HID, HID), lambda p, j: (0, 0)),
            pl.BlockSpec((1, HID), lambda p, j: (0, 0)),
            pl.BlockSpec((1, HID), lambda p, j: (0, 0)),
        ],
        out_specs=pl.BlockSpec((1, NG), lambda p, j: (0, 0)),
        out_shape=jax.ShapeDtypeStruct((1, NG), jnp.float32),
        scratch_shapes=[
            pltpu.VMEM((N, HID), jnp.float32),
            pltpu.VMEM((8, HID), jnp.float32),
            pltpu.VMEM((NG, HID), jnp.float32),
        ],
    )(s_lo, s_hi, W1, b1, W2, b2, gamma, beta, batch3, lin1_W, lin1_b, lin2_r)


def kernel(x, edge_index, batch, params):
    p1 = params["conv1"]
    e2 = edge_index
    CH23, CH1 = 400, 1000
    ep23 = e2.reshape(2, N_EDGES // CH23, CH23).transpose(1, 0, 2)
    ep1 = e2.reshape(2, N_EDGES // CH1, CH1).transpose(1, 0, 2)

    x16 = jnp.pad(x, ((0, 0), (0, D1 - x.shape[1])))
    z16 = jnp.zeros_like(x16)
    batch3 = batch.reshape(NB, 1, BLK)

    layers = [p1] + list(params["convs"])
    out = None
    for li, p in enumerate(layers):
        args = (p["W1"], p["b1"].reshape(1, HID), p["W2"],
                p["b2"].reshape(1, HID), p["gamma"].reshape(1, HID),
                p["beta"].reshape(1, HID))
        if li == 0:
            a0, a1 = _sc_agg1(x16, z16, ep1)
            h_lo, h_hi = _tc_layer(a0, a1, *args, first=True)
        elif li < len(layers) - 1:
            s_lo, s_hi = _sc_agg(h_lo, h_hi, ep23)
            h_lo, h_hi = _tc_layer(s_lo, s_hi, *args, first=False)
        else:
            s_lo, s_hi = _sc_agg(h_lo, h_hi, ep23)
            out = _tc_layer_last(s_lo, s_hi, *args, batch3,
                                 params["lin1_W"],
                                 params["lin1_b"].reshape(1, HID),
                                 params["lin2_W"].reshape(1, HID))

    return out.reshape(NG) + params["lin2_b"][0]


# speedup vs baseline: 1.1358x; 1.0703x over previous
import functools

import jax
import jax.numpy as jnp
from jax import lax
from jax.experimental import pallas as pl
from jax.experimental.pallas import tpu as pltpu
from jax.experimental.pallas import tpu_sc as plsc

N_NODES = 50000
N_EDGES = 800000
HID = 64
HALF = 32
D1 = 16
NG = 512
NC = 2
NS = 16
BLK = 2000
NB = N_NODES // BLK
EPS = 1e-5


def _sc_edge_loop(tab, acc, ep_h, idx0, idx1, rows0, rows1, sem0, sem1,
                  cbase, nch):
    pltpu.sync_copy(ep_h.at[cbase], idx0)
    pltpu.async_copy(tab.at[idx0.at[0]], rows0, sem0)
    pltpu.sync_copy(ep_h.at[cbase + 1], idx1)
    pltpu.async_copy(tab.at[idx1.at[0]], rows1, sem1)

    def pair(g, carry):
        i0 = cbase + 2 * g
        pltpu.make_async_copy(tab.at[idx0.at[0]], rows0, sem0).wait()
        pltpu.sync_copy(rows0, acc.at[idx0.at[1]], add=True)

        @pl.when(2 * g + 2 < nch)
        def _():
            pltpu.sync_copy(ep_h.at[i0 + 2], idx0)
            pltpu.async_copy(tab.at[idx0.at[0]], rows0, sem0)

        @pl.when(2 * g + 1 < nch)
        def _():
            pltpu.make_async_copy(tab.at[idx1.at[0]], rows1, sem1).wait()
            pltpu.sync_copy(rows1, acc.at[idx1.at[1]], add=True)

            @pl.when(2 * g + 3 < nch)
            def _():
                pltpu.sync_copy(ep_h.at[i0 + 3], idx1)
                pltpu.async_copy(tab.at[idx1.at[0]], rows1, sem1)

        return carry

    lax.fori_loop(0, (nch + 1) // 2, pair, 0)


def _sc_agg(h_lo, h_hi, ep):
    N, D = h_lo.shape
    NCHT, _, CH = ep.shape
    NCH = NCHT // NS
    RPT = N // NS

    mesh = plsc.VectorSubcoreMesh(
        core_axis_name="c", subcore_axis_name="s", num_cores=NC, num_subcores=NS
    )

    @functools.partial(
        pl.kernel,
        out_type=(
            jax.ShapeDtypeStruct((N, D), jnp.float32),
            jax.ShapeDtypeStruct((N, D), jnp.float32),
        ),
        mesh=mesh,
        compiler_params=pltpu.CompilerParams(use_tc_tiling_on_sc=False),
        scratch_types=[
            pltpu.VMEM_SHARED((N, D), jnp.float32),
            pltpu.VMEM((2, CH), jnp.int32),
            pltpu.VMEM((2, CH), jnp.int32),
            pltpu.VMEM((CH, D), jnp.float32),
            pltpu.VMEM((CH, D), jnp.float32),
            pltpu.SemaphoreType.DMA,
            pltpu.SemaphoreType.DMA,
        ],
    )
    def k(hlo, hhi, ep_h, olo, ohi,
          acc, idx0, idx1, rows0, rows1, sem0, sem1):
        c = lax.axis_index("c")
        s = lax.axis_index("s")

        def run(tab, out):
            pltpu.sync_copy(tab.at[pl.ds(s * RPT, RPT)],
                            acc.at[pl.ds(s * RPT, RPT)])
            plsc.subcore_barrier()
            _sc_edge_loop(tab, acc, ep_h, idx0, idx1, rows0, rows1,
                          sem0, sem1, s * NCH, NCH)
            plsc.subcore_barrier()
            pltpu.sync_copy(acc.at[pl.ds(s * RPT, RPT)],
                            out.at[pl.ds(s * RPT, RPT)])

        @pl.when(c == 0)
        def _():
            run(hlo, olo)

        @pl.when(c == 1)
        def _():
            run(hhi, ohi)

    return k(h_lo, h_hi, ep)


def _sc_agg1(x16, z16, ep):
    N, D = x16.shape
    NCHT, _, CH = ep.shape
    NCH = NCHT // (NC * NS)
    RPT = N // NS

    mesh = plsc.VectorSubcoreMesh(
        core_axis_name="c", subcore_axis_name="s", num_cores=NC, num_subcores=NS
    )

    @functools.partial(
        pl.kernel,
        out_type=(
            jax.ShapeDtypeStruct((N, D), jnp.float32),
            jax.ShapeDtypeStruct((N, D), jnp.float32),
        ),
        mesh=mesh,
        compiler_params=pltpu.CompilerParams(use_tc_tiling_on_sc=False),
        scratch_types=[
            pltpu.VMEM_SHARED((N, D), jnp.float32),
            pltpu.VMEM((2, CH), jnp.int32),
            pltpu.VMEM((2, CH), jnp.int32),
            pltpu.VMEM((CH, D), jnp.float32),
            pltpu.VMEM((CH, D), jnp.float32),
            pltpu.SemaphoreType.DMA,
            pltpu.SemaphoreType.DMA,
        ],
    )
    def k(x_h, z_h, ep_h, o0, o1,
          acc, idx0, idx1, rows0, rows1, sem0, sem1):
        c = lax.axis_index("c")
        s = lax.axis_index("s")

        def run(init_tab, out):
            pltpu.sync_copy(init_tab.at[pl.ds(s * RPT, RPT)],
                            acc.at[pl.ds(s * RPT, RPT)])
            plsc.subcore_barrier()
            _sc_edge_loop(x_h, acc, ep_h, idx0, idx1, rows0, rows1,
                          sem0, sem1, (c * NS + s) * NCH, NCH)
            plsc.subcore_barrier()
            pltpu.sync_copy(acc.at[pl.ds(s * RPT, RPT)],
                            out.at[pl.ds(s * RPT, RPT)])

        @pl.when(c == 0)
        def _():
            run(x_h, o0)

        @pl.when(c == 1)
        def _():
            run(z_h, o1)

    return k(x16, z16, ep)


def _bn_coeffs(st, gamma, beta):
    mean = st[0:1, :] / N_NODES
    var = st[1:2, :] / N_NODES - mean * mean
    rstd = gamma / jnp.sqrt(var + EPS)
    return rstd, beta - mean * rstd


def _tc_layer(s_lo, s_hi, W1, b1, W2, b2, gamma, beta, first):
    N = s_lo.shape[0]

    def body(lo_ref, hi_ref, w1_ref, b1_ref, w2_ref, b2_ref, g_ref, be_ref,
             olo_ref, ohi_ref, u_scr, st_scr):
        p = pl.program_id(0)
        j = pl.program_id(1)

        @pl.when(p == 0)
        def _():
            if first:
                s = (lo_ref[...] + hi_ref[...])[:, 0:2]
            else:
                s = jnp.concatenate([lo_ref[...], hi_ref[...]], axis=1)
            a = jnp.dot(s, w1_ref[...], preferred_element_type=jnp.float32)
            a = jnp.maximum(a + b1_ref[...], 0.0)
            b = jnp.dot(a, w2_ref[...], preferred_element_type=jnp.float32)
            b = jnp.maximum(b + b2_ref[...], 0.0)
            u_scr[pl.ds(j * BLK, BLK), :] = b

            @pl.when(j == 0)
            def _():
                st_scr[...] = jnp.zeros_like(st_scr)

            s0 = jnp.sum(b, axis=0, keepdims=True)
            s1 = jnp.sum(b * b, axis=0, keepdims=True)
            pad = jnp.zeros((6, HID), jnp.float32)
            st_scr[...] = st_scr[...] + jnp.concatenate([s0, s1, pad], axis=0)

        @pl.when(p == 1)
        def _():
            scale, shift = _bn_coeffs(st_scr[...], g_ref[...], be_ref[...])
            hn = u_scr[pl.ds(j * BLK, BLK), :] * scale + shift
            olo_ref[...] = hn[:, :HALF]
            ohi_ref[...] = hn[:, HALF:]

    din = s_lo.shape[1]
    return pl.pallas_call(
        body,
        grid=(2, NB),
        in_specs=[
            pl.BlockSpec((BLK, din), lambda p, j: (j * (1 - p), 0)),
            pl.BlockSpec((BLK, din), lambda p, j: (j * (1 - p), 0)),
            pl.BlockSpec(W1.shape, lambda p, j: (0, 0)),
            pl.BlockSpec((1, HID), lambda p, j: (0, 0)),
            pl.BlockSpec((HID, HID), lambda p, j: (0, 0)),
            pl.BlockSpec((1, HID), lambda p, j: (0, 0)),
            pl.BlockSpec((1, HID), lambda p, j: (0, 0)),
            pl.BlockSpec((1, HID), lambda p, j: (0, 0)),
        ],
        out_specs=[pl.BlockSpec((BLK, HALF), lambda p, j: (j * p, 0))] * 2,
        out_shape=[jax.ShapeDtypeStruct((N, HALF), jnp.float32)] * 2,
        scratch_shapes=[
            pltpu.VMEM((N, HID), jnp.float32),
            pltpu.VMEM((8, HID), jnp.float32),
        ],
    )(s_lo, s_hi, W1, b1, W2, b2, gamma, beta)


def _tc_layer_last(s_lo, s_hi, W1, b1, W2, b2, gamma, beta, batch3,
                   lin1_W, lin1_b, lin2_r):
    N = s_lo.shape[0]

    def body(lo_ref, hi_ref, w1_ref, b1_ref, w2_ref, b2_ref, g_ref, be_ref,
             b3_ref, lw_ref, lb_ref, l2_ref, out_ref, u_scr, st_scr, acc_ref):
        p = pl.program_id(0)
        j = pl.program_id(1)

        @pl.when(p == 0)
        def _():
            s = jnp.concatenate([lo_ref[...], hi_ref[...]], axis=1)
            a = jnp.dot(s, w1_ref[...], preferred_element_type=jnp.float32)
            a = jnp.maximum(a + b1_ref[...], 0.0)
            b = jnp.dot(a, w2_ref[...], preferred_element_type=jnp.float32)
            b = jnp.maximum(b + b2_ref[...], 0.0)
            u_scr[pl.ds(j * BLK, BLK), :] = b

            @pl.when(j == 0)
            def _():
                st_scr[...] = jnp.zeros_like(st_scr)

            s0 = jnp.sum(b, axis=0, keepdims=True)
            s1 = jnp.sum(b * b, axis=0, keepdims=True)
            pad = jnp.zeros((6, HID), jnp.float32)
            st_scr[...] = st_scr[...] + jnp.concatenate([s0, s1, pad], axis=0)

        @pl.when(p == 1)
        def _():
            scale, shift = _bn_coeffs(st_scr[...], g_ref[...], be_ref[...])
            hn = u_scr[pl.ds(j * BLK, BLK), :] * scale + shift
            bb = b3_ref[0, 0, :]
            seg = jax.lax.broadcasted_iota(jnp.int32, (BLK, NG), 1)
            onehot = (bb[:, None] == seg).astype(jnp.float32)
            part = jax.lax.dot_general(
                onehot, hn, (((0,), (0,)), ((), ())),
                preferred_element_type=jnp.float32)

            @pl.when(j == 0)
            def _():
                acc_ref[...] = jnp.zeros_like(acc_ref)

            acc_ref[...] = acc_ref[...] + part

            @pl.when(j == NB - 1)
            def _():
                pooled = acc_ref[...]
                r = jnp.dot(pooled, lw_ref[...],
                            preferred_element_type=jnp.float32)
                r = jnp.maximum(r + lb_ref[...], 0.0)
                o = jnp.sum(r * l2_ref[...], axis=1)
                out_ref[...] = o.reshape(1, NG)

    return pl.pallas_call(
        body,
        grid=(2, NB),
        in_specs=[
            pl.BlockSpec((BLK, HALF), lambda p, j: (j * (1 - p), 0)),
            pl.BlockSpec((BLK, HALF), lambda p, j: (j * (1 - p), 0)),
            pl.BlockSpec((HID, HID), lambda p, j: (0, 0)),
            pl.BlockSpec((1, HID), lambda p, j: (0, 0)),
            pl.BlockSpec((HID, HID), lambda p, j: (0, 0)),
            pl.BlockSpec((1, HID), lambda p, j: (0, 0)),
            pl.BlockSpec((1, HID), lambda p, j: (0, 0)),
            pl.BlockSpec((1, HID), lambda p, j: (0, 0)),
            pl.BlockSpec((1, 1, BLK), lambda p, j: (j * p, 0, 0)),
            pl.BlockSpec((HID, HID), lambda p, j: (0, 0)),
            pl.BlockSpec((1, HID), lambda p, j: (0, 0)),
            pl.BlockSpec((1, HID), lambda p, j: (0, 0)),
        ],
        out_specs=pl.BlockSpec((1, NG), lambda p, j: (0, 0)),
        out_shape=jax.ShapeDtypeStruct((1, NG), jnp.float32),
        scratch_shapes=[
            pltpu.VMEM((N, HID), jnp.float32),
            pltpu.VMEM((8, HID), jnp.float32),
            pltpu.VMEM((NG, HID), jnp.float32),
        ],
    )(s_lo, s_hi, W1, b1, W2, b2, gamma, beta, batch3, lin1_W, lin1_b, lin2_r)


def kernel(x, edge_index, batch, params):
    p1 = params["conv1"]
    e2 = edge_index
    CH23, CH1 = 400, 1000
    ep23 = e2.reshape(2, N_EDGES // CH23, CH23).transpose(1, 0, 2)
    ep1 = e2.reshape(2, N_EDGES // CH1, CH1).transpose(1, 0, 2)

    x16 = jnp.pad(x, ((0, 0), (0, D1 - x.shape[1])))
    z16 = jnp.zeros_like(x16)
    batch3 = batch.reshape(NB, 1, BLK)

    layers = [p1] + list(params["convs"])
    out = None
    for li, p in enumerate(layers):
        args = (p["W1"], p["b1"].reshape(1, HID), p["W2"],
                p["b2"].reshape(1, HID), p["gamma"].reshape(1, HID),
                p["beta"].reshape(1, HID))
        if li == 0:
            a0, a1 = _sc_agg1(x16, z16, ep1)
            h_lo, h_hi = _tc_layer(a0, a1, *args, first=True)
        elif li < len(layers) - 1:
            s_lo, s_hi = _sc_agg(h_lo, h_hi, ep23)
            h_lo, h_hi = _tc_layer(s_lo, s_hi, *args, first=False)
        else:
            s_lo, s_hi = _sc_agg(h_lo, h_hi, ep23)
            out = _tc_layer_last(s_lo, s_hi, *args, batch3,
                                 params["lin1_W"],
                                 params["lin1_b"].reshape(1, HID),
                                 params["lin2_W"].reshape(1, HID))

    return out.reshape(NG) + params["lin2_b"][0]
